# Initial kernel scaffold; baseline (speedup 1.0000x reference)
import jax
import jax.numpy as jnp
from jax.experimental import pallas as pl


def kernel(z, feature1, feature2, edge_index, batch, params):
    raise NotImplementedError("write your pallas kernel here")


# speedup vs baseline: 1.7821x; 1.7821x over previous
import functools

import jax
import jax.numpy as jnp
from jax import lax
from jax.experimental import pallas as pl
from jax.experimental.pallas import tpu as pltpu
from jax.experimental.pallas import tpu_sc as plsc

N = 10000
E = 160000
H = 256
HH = H // 2
NG = 64
F1 = 12
F2 = 6

NC = 2
NS = 16
NW = NC * NS
CH = 128
EP = 163840
GCP = EP // NW // CH
SCP = EP // NS // CH
NP = 10240
ZR = 128
RPS = NP // NS


def _rdot(a, b):
    return jnp.dot(a, b, preferred_element_type=jnp.float32)


def _dot(a, b):
    ah = a.astype(jnp.bfloat16).astype(jnp.float32)
    al = a - ah
    bh = b.astype(jnp.bfloat16).astype(jnp.float32)
    bl = b - bh
    return _rdot(ah, bh) + (_rdot(al, bh) + _rdot(ah, bl))


def _dot1(oh, b):
    bh = b.astype(jnp.bfloat16).astype(jnp.float32)
    return _rdot(oh, bh) + _rdot(oh, b - bh)


_DNT = (((0,), (0,)), ((), ()))


def _dotT(a, b):
    bh = b.astype(jnp.bfloat16).astype(jnp.float32)
    return (lax.dot_general(a, bh, _DNT, preferred_element_type=jnp.float32)
            + lax.dot_general(a, b - bh, _DNT,
                              preferred_element_type=jnp.float32))


def _swish(v):
    return v / (1.0 + jnp.exp(-v))


_TN = 2000


def _prologue(z, emb):
    def body(z_ref, emb_ref, x_ref):
        zz = z_ref[...].reshape(_TN, 1)
        oh = (zz == lax.broadcasted_iota(jnp.int32, (_TN, 95), 1)).astype(jnp.float32)
        x_ref[...] = _swish(_dot1(oh, emb_ref[...]))

    return pl.pallas_call(
        body,
        grid=(N // _TN,),
        in_specs=[pl.BlockSpec((_TN, 1), lambda i: (i, 0)),
                  pl.BlockSpec((95, H), lambda i: (0, 0))],
        out_specs=pl.BlockSpec((_TN, H), lambda i: (i, 0)),
        out_shape=jax.ShapeDtypeStruct((N, H), jnp.float32))(z, emb)


def _stage_a(x, lin_Wt, lin_b, f1_W1t, f1_W2t, f2_W1t, f2_W2t):
    def body(x_ref, w_ref, b_ref, a1_ref, b1_ref, a2_ref, b2_ref,
             xo_ref, c1_ref, c2_ref):
        i = pl.program_id(0)
        h = _dot(x_ref[...], w_ref[...])
        xo_ref[...] = _swish(h + b_ref[...][None, :])

        @pl.when(i == 0)
        def _():
            c1_ref[...] = _dot(a1_ref[...], b1_ref[...])
            c2_ref[...] = _dot(a2_ref[...], b2_ref[...])

    wfix = lambda shape: pl.BlockSpec(shape, lambda i: tuple(0 for _ in shape))
    return pl.pallas_call(
        body,
        grid=(N // _TN,),
        in_specs=[pl.BlockSpec((_TN, H), lambda i: (i, 0)),
                  wfix((H, H)), wfix((H,)),
                  wfix((F1, 64)), wfix((64, H)),
                  wfix((F2, 64)), wfix((64, H))],
        out_specs=[pl.BlockSpec((_TN, H), lambda i: (i, 0)),
                   wfix((F1, H)), wfix((F2, H))],
        out_shape=(jax.ShapeDtypeStruct((N, H), jnp.float32),
                   jax.ShapeDtypeStruct((F1, H), jnp.float32),
                   jax.ShapeDtypeStruct((F2, H), jnp.float32)),
    )(x, lin_Wt, lin_b, f1_W1t, f1_W2t, f2_W1t, f2_W2t)


_TE = 2048


def _stage_b(f1p, f2p, xg, c1t, c2t):
    def body(f1_ref, f2_ref, xg_ref, c1_ref, c2_ref, m1_ref, m2_ref):
        xgv = xg_ref[...]
        f1h = _dot(f1_ref[...], c1_ref[...])
        f2h = _dot(f2_ref[...], c2_ref[...])
        m1 = f1h * xgv
        m2 = f2h * xgv
        m1_ref[0] = m1[:, :HH]
        m1_ref[1] = m1[:, HH:]
        m2_ref[0] = m2[:, :HH]
        m2_ref[1] = m2[:, HH:]

    return pl.pallas_call(
        body,
        grid=(EP // _TE,),
        in_specs=[pl.BlockSpec((_TE, F1), lambda i: (i, 0)),
                  pl.BlockSpec((_TE, F2), lambda i: (i, 0)),
                  pl.BlockSpec((_TE, H), lambda i: (i, 0)),
                  pl.BlockSpec((F1, H), lambda i: (0, 0)),
                  pl.BlockSpec((F2, H), lambda i: (0, 0))],
        out_specs=[pl.BlockSpec((NC, _TE, HH), lambda i: (0, i, 0)),
                   pl.BlockSpec((NC, _TE, HH), lambda i: (0, i, 0))],
        out_shape=(jax.ShapeDtypeStruct((NC, EP, HH), jnp.float32),
                   jax.ShapeDtypeStruct((NC, EP, HH), jnp.float32)),
    )(f1p, f2p, xg, c1t, c2t)


def _stage_c1(xnew, a1, a2, w):
    def body(x_ref, a1_ref, a2_ref,
             wrel1, brel1, wroot1, wlin1, blin1,
             wrel2, brel2, wroot2, wlin2, blin2,
             catl, catr, catb,
             l0w, l0b, l1w, l1b, l2w, l2b, o_ref):
        x = x_ref[...]
        a1v = jnp.concatenate([a1_ref[0], a1_ref[1]], axis=1)
        a2v = jnp.concatenate([a2_ref[0], a2_ref[1]], axis=1)
        h1 = (_dot(a1v, wrel1[...])
              + brel1[...][None, :]
              + _dot(x, wroot1[...]))
        h1 = _swish(_dot(h1, wlin1[...])
                    + blin1[...][None, :])
        h2 = (_dot(a2v, wrel2[...])
              + brel2[...][None, :]
              + _dot(x, wroot2[...]))
        h2 = _swish(_dot(h2, wlin2[...])
                    + blin2[...][None, :])
        h = (_dot(h1, catl[...])
             + _dot(h2, catr[...])
             + catb[...][None, :])
        h = h + x
        for wref, bref in ((l0w, l0b), (l1w, l1b), (l2w, l2b)):
            h = _swish(_dot(h, wref[...])
                       + bref[...][None, :]) + h
        o_ref[...] = h

    wfull = pl.BlockSpec((H, H), lambda i: (0, 0))
    wvec = pl.BlockSpec((H,), lambda i: (0,))
    return pl.pallas_call(
        body,
        grid=(N // _TN,),
        in_specs=[pl.BlockSpec((_TN, H), lambda i: (i, 0)),
                  pl.BlockSpec((NC, _TN, HH), lambda i: (0, i, 0)),
                  pl.BlockSpec((NC, _TN, HH), lambda i: (0, i, 0)),
                  wfull, wvec, wfull, wfull, wvec,
                  wfull, wvec, wfull, wfull, wvec,
                  wfull, wfull, wvec,
                  wfull, wvec, wfull, wvec, wfull, wvec],
        out_specs=pl.BlockSpec((_TN, H), lambda i: (i, 0)),
        out_shape=jax.ShapeDtypeStruct((N, H), jnp.float32),
    )(xnew, a1, a2, *w)


def _onehot(bt, rows):
    return (bt.reshape(rows, 1)
            == lax.broadcasted_iota(jnp.int32, (rows, NG), 1)).astype(jnp.float32)


def _stage_c2(h, batch, nw, nb, na, fw, fb):
    bt = batch

    def body_a(h_ref, bt_ref, s_ref, c_ref):
        i = pl.program_id(0)
        Bm = _onehot(bt_ref[...], _TN)

        @pl.when(i == 0)
        def _():
            s_ref[...] = jnp.zeros_like(s_ref)
            c_ref[...] = jnp.zeros_like(c_ref)

        s_ref[...] += _dotT(Bm, h_ref[...])
        c_ref[...] += lax.dot_general(Bm, jnp.ones((_TN, 1), jnp.float32),
                                      _DNT, preferred_element_type=jnp.float32)

    s, cnt = pl.pallas_call(
        body_a,
        grid=(N // _TN,),
        in_specs=[pl.BlockSpec((_TN, H), lambda i: (i, 0)),
                  pl.BlockSpec((_TN, 1), lambda i: (i, 0))],
        out_specs=[pl.BlockSpec((NG, H), lambda i: (0, 0)),
                   pl.BlockSpec((NG, 1), lambda i: (0, 0))],
        out_shape=(jax.ShapeDtypeStruct((NG, H), jnp.float32),
                   jax.ShapeDtypeStruct((NG, 1), jnp.float32)),
    )(h, bt)

    def body_b(h_ref, bt_ref, s_ref, c_ref, na_ref, o_ref, v_ref):
        i = pl.program_id(0)
        Bm = _onehot(bt_ref[...], _TN)
        cnt = jnp.maximum(c_ref[...], 1.0)
        mean = s_ref[...] / cnt
        out = h_ref[...] - na_ref[...][None, :] * _dot1(Bm, mean)
        o_ref[...] = out

        @pl.when(i == 0)
        def _():
            v_ref[...] = jnp.zeros_like(v_ref)

        v_ref[...] += _dotT(Bm, out * out)

    out, v = pl.pallas_call(
        body_b,
        grid=(N // _TN,),
        in_specs=[pl.BlockSpec((_TN, H), lambda i: (i, 0)),
                  pl.BlockSpec((_TN, 1), lambda i: (i, 0)),
                  pl.BlockSpec((NG, H), lambda i: (0, 0)),
                  pl.BlockSpec((NG, 1), lambda i: (0, 0)),
                  pl.BlockSpec((H,), lambda i: (0,))],
        out_specs=[pl.BlockSpec((_TN, H), lambda i: (i, 0)),
                   pl.BlockSpec((NG, H), lambda i: (0, 0))],
        out_shape=(jax.ShapeDtypeStruct((N, H), jnp.float32),
                   jax.ShapeDtypeStruct((NG, H), jnp.float32)),
    )(h, bt, s, cnt, na)

    def body_c(o_ref, bt_ref, v_ref, c_ref, nw_ref, nb_ref, fw_ref, fb_ref,
               x_ref):
        Bm = _onehot(bt_ref[...], _TN)
        cnt = jnp.maximum(c_ref[...], 1.0)
        isd = lax.rsqrt(v_ref[...] / cnt + 1e-5)
        outf = o_ref[...] * _dot1(Bm, isd)
        hn = nw_ref[...][None, :] * outf + nb_ref[...][None, :]
        x_ref[...] = _dot(hn, fw_ref[...]) + fb_ref[...][None, :]

    return pl.pallas_call(
        body_c,
        grid=(N // _TN,),
        in_specs=[pl.BlockSpec((_TN, H), lambda i: (i, 0)),
                  pl.BlockSpec((_TN, 1), lambda i: (i, 0)),
                  pl.BlockSpec((NG, H), lambda i: (0, 0)),
                  pl.BlockSpec((NG, 1), lambda i: (0, 0)),
                  pl.BlockSpec((H,), lambda i: (0,)),
                  pl.BlockSpec((H,), lambda i: (0,)),
                  pl.BlockSpec((H, H), lambda i: (0, 0)),
                  pl.BlockSpec((H,), lambda i: (0,))],
        out_specs=pl.BlockSpec((_TN, H), lambda i: (i, 0)),
        out_shape=jax.ShapeDtypeStruct((N, H), jnp.float32),
    )(out, bt, v, cnt, nw, nb, fw, fb)


def _stage_d(x, batch, w):
    def body(x_ref, bt_ref, w0, b0, w1, b1, w2, b2, ow, ob, o_ref):
        i = pl.program_id(0)
        x = x_ref[...]
        for wref, bref in ((w0, b0), (w1, b1), (w2, b2)):
            x = _swish(_dot(x, wref[...])
                       + bref[...][None, :])
        xo = _dot(x, ow[...]) + ob[...][None, :]
        Bm = _onehot(bt_ref[...], _TN)

        @pl.when(i == 0)
        def _():
            o_ref[...] = jnp.zeros_like(o_ref)

        o_ref[...] += _dotT(Bm, xo)

    wfull = pl.BlockSpec((H, H), lambda i: (0, 0))
    wvec = pl.BlockSpec((H,), lambda i: (0,))
    return pl.pallas_call(
        body,
        grid=(N // _TN,),
        in_specs=[pl.BlockSpec((_TN, H), lambda i: (i, 0)),
                  pl.BlockSpec((_TN, 1), lambda i: (i, 0)),
                  wfull, wvec, wfull, wvec, wfull, wvec,
                  pl.BlockSpec((H, 1), lambda i: (0, 0)),
                  pl.BlockSpec((1,), lambda i: (0,))],
        out_specs=pl.BlockSpec((NG, 1), lambda i: (0, 0)),
        out_shape=jax.ShapeDtypeStruct((NG, 1), jnp.float32),
    )(x, batch, *w)


def _sc_gather(x, src3):
    mesh = plsc.VectorSubcoreMesh(core_axis_name="c", subcore_axis_name="s")

    @functools.partial(
        pl.kernel, mesh=mesh,
        out_type=jax.ShapeDtypeStruct((EP, H), jnp.float32),
        scratch_types=[
            pltpu.VMEM((CH,), jnp.int32),
            pltpu.VMEM((CH, H), jnp.float32),
            pltpu.SemaphoreType.DMA,
        ],
    )
    def k(x_hbm, src_hbm, out_hbm, idx_v, rows_v, sem):
        wid = lax.axis_index("s") * NC + lax.axis_index("c")

        def body(i, carry):
            pltpu.sync_copy(src_hbm.at[wid, i], idx_v)
            pltpu.async_copy(x_hbm.at[idx_v], rows_v, sem).wait()
            base = (wid * GCP + i) * CH
            pltpu.sync_copy(rows_v, out_hbm.at[pl.ds(base, CH)])
            return carry

        lax.fori_loop(0, GCP, body, 0)

    return k(x, src3)


def _sc_scatter(m, dst3):
    mesh = plsc.VectorSubcoreMesh(core_axis_name="c", subcore_axis_name="s")

    @functools.partial(
        pl.kernel, mesh=mesh,
        out_type=jax.ShapeDtypeStruct((NC, NP, HH), jnp.float32),
        scratch_types=[
            pltpu.VMEM((CH,), jnp.int32),
            pltpu.VMEM((CH, HH), jnp.float32),
            pltpu.VMEM((ZR, HH), jnp.float32),
            pltpu.VMEM_SHARED((NP, HH), jnp.float32),
        ],
    )
    def k(m_hbm, dst_hbm, out_hbm, idx_v, buf_v, st_v, acc_sh):
        c = lax.axis_index("c")
        s = lax.axis_index("s")

        def zrow(i, carry):
            def zcol(j, carry2):
                st_v[i, pl.ds(j * 16, 16)] = jnp.zeros((16,), jnp.float32)
                return carry2
            return lax.fori_loop(0, HH // 16, zcol, carry)

        lax.fori_loop(0, ZR, zrow, 0)
        for r in range(RPS // ZR):
            pltpu.sync_copy(st_v, acc_sh.at[pl.ds(s * RPS + r * ZR, ZR)])
        plsc.subcore_barrier()

        def body(i, carry):
            pltpu.sync_copy(dst_hbm.at[s, i], idx_v)
            base = (s * SCP + i) * CH
            pltpu.sync_copy(m_hbm.at[c, pl.ds(base, CH)], buf_v)
            pltpu.sync_copy(buf_v, acc_sh.at[idx_v], add=True)
            return carry

        lax.fori_loop(0, SCP, body, 0)
        plsc.subcore_barrier()

        for r in range(RPS // ZR):
            row = s * RPS + r * ZR
            pltpu.sync_copy(acc_sh.at[pl.ds(row, ZR)], st_v)
            pltpu.sync_copy(st_v, out_hbm.at[c, pl.ds(row, ZR)])

    return k(m, dst3)


def kernel(z, feature1, feature2, edge_index, batch, params):
    src = edge_index[0].astype(jnp.int32)
    dst = edge_index[1].astype(jnp.int32)
    pad = EP - E
    src3 = jnp.pad(src, (0, pad)).reshape(NW, GCP, CH)
    dst3 = jnp.pad(dst, (0, pad)).reshape(NS, SCP, CH)
    f1p = jnp.pad(feature1, ((0, pad), (0, 0)))
    f2p = jnp.pad(feature2, ((0, pad), (0, 0)))

    bt2 = batch.astype(jnp.int32).reshape(N, 1)
    x = _prologue(z.astype(jnp.int32).reshape(N, 1), params['emb'])

    for b in params['blocks']:
        xnew, c1t, c2t = _stage_a(
            x, b['lin_W'].T, b['lin_b'],
            b['f1_W1'].T, b['f1_W2'].T, b['f2_W1'].T, b['f2_W2'].T)
        xg = _sc_gather(xnew, src3)
        m1, m2 = _stage_b(f1p, f2p, xg, c1t, c2t)
        a1 = _sc_scatter(m1, dst3)[:, :N, :]
        a2 = _sc_scatter(m2, dst3)[:, :N, :]
        wlist = [
            b['c1_Wrel'].T, b['c1_brel'], b['c1_Wroot'].T,
            b['lin1_W'].T, b['lin1_b'],
            b['c2_Wrel'].T, b['c2_brel'], b['c2_Wroot'].T,
            b['lin2_W'].T, b['lin2_b'],
            b['cat_W'][:, :H].T, b['cat_W'][:, H:].T, b['cat_b'],
            b['lins'][0][0].T, b['lins'][0][1],
            b['lins'][1][0].T, b['lins'][1][1],
            b['lins'][2][0].T, b['lins'][2][1],
        ]
        h = _stage_c1(xnew, a1, a2, wlist)
        x = _stage_c2(h, bt2,
                      b['norm_w'], b['norm_b'], b['norm_a'],
                      b['final_W'].T, b['final_b'])

    wout = [
        params['out_lins'][0][0].T, params['out_lins'][0][1],
        params['out_lins'][1][0].T, params['out_lins'][1][1],
        params['out_lins'][2][0].T, params['out_lins'][2][1],
        params['out_W'].T, params['out_b'],
    ]
    return _stage_d(x, bt2, wout)


# speedup vs baseline: 1.8743x; 1.0517x over previous
import functools

import jax
import jax.numpy as jnp
from jax import lax
from jax.experimental import pallas as pl
from jax.experimental.pallas import tpu as pltpu
from jax.experimental.pallas import tpu_sc as plsc

N = 10000
E = 160000
H = 256
HH = H // 2
NG = 64
F1 = 12
F2 = 6

NC = 2
NS = 16
NW = NC * NS
CH = 128
EP = 163840
GCP = EP // NW // CH
SCP = EP // NS // CH
NP = 10240
ZR = 128
RPS = NP // NS


def _rdot(a, b):
    return jnp.dot(a, b, preferred_element_type=jnp.float32)


def _dot(a, b):
    ah = a.astype(jnp.bfloat16).astype(jnp.float32)
    al = a - ah
    bh = b.astype(jnp.bfloat16).astype(jnp.float32)
    bl = b - bh
    return _rdot(ah, bh) + (_rdot(al, bh) + _rdot(ah, bl))


def _dot1(oh, b):
    bh = b.astype(jnp.bfloat16).astype(jnp.float32)
    return _rdot(oh, bh) + _rdot(oh, b - bh)


_DNT = (((0,), (0,)), ((), ()))


def _dotT(a, b):
    bh = b.astype(jnp.bfloat16).astype(jnp.float32)
    return (lax.dot_general(a, bh, _DNT, preferred_element_type=jnp.float32)
            + lax.dot_general(a, b - bh, _DNT,
                              preferred_element_type=jnp.float32))


def _swish(v):
    return v / (1.0 + jnp.exp(-v))


_TN = 2000


def _prologue(z, emb):
    def body(z_ref, emb_ref, x_ref):
        zz = z_ref[...].reshape(_TN, 1)
        oh = (zz == lax.broadcasted_iota(jnp.int32, (_TN, 95), 1)).astype(jnp.float32)
        x_ref[...] = _swish(_dot1(oh, emb_ref[...]))

    return pl.pallas_call(
        body,
        grid=(N // _TN,),
        in_specs=[pl.BlockSpec((_TN, 1), lambda i: (i, 0)),
                  pl.BlockSpec((95, H), lambda i: (0, 0))],
        out_specs=pl.BlockSpec((_TN, H), lambda i: (i, 0)),
        out_shape=jax.ShapeDtypeStruct((N, H), jnp.float32))(z, emb)


def _stage_a(x, lin_Wt, lin_b, f1_W1t, f1_W2t, f2_W1t, f2_W2t):
    def body(x_ref, w_ref, b_ref, a1_ref, b1_ref, a2_ref, b2_ref,
             xo_ref, xb_ref, c1_ref, c2_ref):
        i = pl.program_id(0)
        h = _dot(x_ref[...], w_ref[...])
        xo = _swish(h + b_ref[...][None, :])
        xo_ref[...] = xo
        xb_ref[...] = xo.astype(jnp.bfloat16)

        @pl.when(i == 0)
        def _():
            c1_ref[...] = _dot(a1_ref[...], b1_ref[...])
            c2_ref[...] = _dot(a2_ref[...], b2_ref[...])

    wfix = lambda shape: pl.BlockSpec(shape, lambda i: tuple(0 for _ in shape))
    return pl.pallas_call(
        body,
        grid=(N // _TN,),
        in_specs=[pl.BlockSpec((_TN, H), lambda i: (i, 0)),
                  wfix((H, H)), wfix((H,)),
                  wfix((F1, 64)), wfix((64, H)),
                  wfix((F2, 64)), wfix((64, H))],
        out_specs=[pl.BlockSpec((_TN, H), lambda i: (i, 0)),
                   pl.BlockSpec((_TN, H), lambda i: (i, 0)),
                   wfix((F1, H)), wfix((F2, H))],
        out_shape=(jax.ShapeDtypeStruct((N, H), jnp.float32),
                   jax.ShapeDtypeStruct((N, H), jnp.bfloat16),
                   jax.ShapeDtypeStruct((F1, H), jnp.float32),
                   jax.ShapeDtypeStruct((F2, H), jnp.float32)),
    )(x, lin_Wt, lin_b, f1_W1t, f1_W2t, f2_W1t, f2_W2t)


_TE = 2048


def _stage_b(f1p, f2p, xg, c1t, c2t):
    def body(f1_ref, f2_ref, xg_ref, c1_ref, c2_ref, m1_ref, m2_ref):
        xgp = xg_ref[...]
        xlo = lax.bitcast_convert_type(xgp << 16, jnp.float32)
        xhi = lax.bitcast_convert_type(xgp & jnp.int32(-65536), jnp.float32)
        f1h = _dot(f1_ref[...], c1_ref[...])
        f2h = _dot(f2_ref[...], c2_ref[...])
        m1_ref[0] = f1h[:, :HH] * xlo
        m1_ref[1] = f1h[:, HH:] * xhi
        m2_ref[0] = f2h[:, :HH] * xlo
        m2_ref[1] = f2h[:, HH:] * xhi

    return pl.pallas_call(
        body,
        grid=(EP // _TE,),
        in_specs=[pl.BlockSpec((_TE, F1), lambda i: (i, 0)),
                  pl.BlockSpec((_TE, F2), lambda i: (i, 0)),
                  pl.BlockSpec((_TE, HH), lambda i: (i, 0)),
                  pl.BlockSpec((F1, H), lambda i: (0, 0)),
                  pl.BlockSpec((F2, H), lambda i: (0, 0))],
        out_specs=[pl.BlockSpec((NC, _TE, HH), lambda i: (0, i, 0)),
                   pl.BlockSpec((NC, _TE, HH), lambda i: (0, i, 0))],
        out_shape=(jax.ShapeDtypeStruct((NC, EP, HH), jnp.float32),
                   jax.ShapeDtypeStruct((NC, EP, HH), jnp.float32)),
    )(f1p, f2p, xg, c1t, c2t)


def _stage_c1(xnew, a1, a2, w):
    def body(x_ref, a1_ref, a2_ref,
             wrel1, brel1, wroot1, wlin1, blin1,
             wrel2, brel2, wroot2, wlin2, blin2,
             catl, catr, catb,
             l0w, l0b, l1w, l1b, l2w, l2b, o_ref):
        x = x_ref[...]
        a1v = jnp.concatenate([a1_ref[0], a1_ref[1]], axis=1)
        a2v = jnp.concatenate([a2_ref[0], a2_ref[1]], axis=1)
        h1 = (_dot(a1v, wrel1[...])
              + brel1[...][None, :]
              + _dot(x, wroot1[...]))
        h1 = _swish(_dot(h1, wlin1[...])
                    + blin1[...][None, :])
        h2 = (_dot(a2v, wrel2[...])
              + brel2[...][None, :]
              + _dot(x, wroot2[...]))
        h2 = _swish(_dot(h2, wlin2[...])
                    + blin2[...][None, :])
        h = (_dot(h1, catl[...])
             + _dot(h2, catr[...])
             + catb[...][None, :])
        h = h + x
        for wref, bref in ((l0w, l0b), (l1w, l1b), (l2w, l2b)):
            h = _swish(_dot(h, wref[...])
                       + bref[...][None, :]) + h
        o_ref[...] = h

    wfull = pl.BlockSpec((H, H), lambda i: (0, 0))
    wvec = pl.BlockSpec((H,), lambda i: (0,))
    return pl.pallas_call(
        body,
        grid=(N // _TN,),
        in_specs=[pl.BlockSpec((_TN, H), lambda i: (i, 0)),
                  pl.BlockSpec((NC, _TN, HH), lambda i: (0, i, 0)),
                  pl.BlockSpec((NC, _TN, HH), lambda i: (0, i, 0)),
                  wfull, wvec, wfull, wfull, wvec,
                  wfull, wvec, wfull, wfull, wvec,
                  wfull, wfull, wvec,
                  wfull, wvec, wfull, wvec, wfull, wvec],
        out_specs=pl.BlockSpec((_TN, H), lambda i: (i, 0)),
        out_shape=jax.ShapeDtypeStruct((N, H), jnp.float32),
    )(xnew, a1, a2, *w)


def _onehot(bt, rows):
    return (bt.reshape(rows, 1)
            == lax.broadcasted_iota(jnp.int32, (rows, NG), 1)).astype(jnp.float32)


def _stage_c2(h, batch, nw, nb, na, fw, fb):
    bt = batch

    def body_a(h_ref, bt_ref, s_ref, c_ref):
        i = pl.program_id(0)
        Bm = _onehot(bt_ref[...], _TN)

        @pl.when(i == 0)
        def _():
            s_ref[...] = jnp.zeros_like(s_ref)
            c_ref[...] = jnp.zeros_like(c_ref)

        s_ref[...] += _dotT(Bm, h_ref[...])
        c_ref[...] += lax.dot_general(Bm, jnp.ones((_TN, 1), jnp.float32),
                                      _DNT, preferred_element_type=jnp.float32)

    s, cnt = pl.pallas_call(
        body_a,
        grid=(N // _TN,),
        in_specs=[pl.BlockSpec((_TN, H), lambda i: (i, 0)),
                  pl.BlockSpec((_TN, 1), lambda i: (i, 0))],
        out_specs=[pl.BlockSpec((NG, H), lambda i: (0, 0)),
                   pl.BlockSpec((NG, 1), lambda i: (0, 0))],
        out_shape=(jax.ShapeDtypeStruct((NG, H), jnp.float32),
                   jax.ShapeDtypeStruct((NG, 1), jnp.float32)),
    )(h, bt)

    def body_b(h_ref, bt_ref, s_ref, c_ref, na_ref, o_ref, v_ref):
        i = pl.program_id(0)
        Bm = _onehot(bt_ref[...], _TN)
        cnt = jnp.maximum(c_ref[...], 1.0)
        mean = s_ref[...] / cnt
        out = h_ref[...] - na_ref[...][None, :] * _dot1(Bm, mean)
        o_ref[...] = out

        @pl.when(i == 0)
        def _():
            v_ref[...] = jnp.zeros_like(v_ref)

        v_ref[...] += _dotT(Bm, out * out)

    out, v = pl.pallas_call(
        body_b,
        grid=(N // _TN,),
        in_specs=[pl.BlockSpec((_TN, H), lambda i: (i, 0)),
                  pl.BlockSpec((_TN, 1), lambda i: (i, 0)),
                  pl.BlockSpec((NG, H), lambda i: (0, 0)),
                  pl.BlockSpec((NG, 1), lambda i: (0, 0)),
                  pl.BlockSpec((H,), lambda i: (0,))],
        out_specs=[pl.BlockSpec((_TN, H), lambda i: (i, 0)),
                   pl.BlockSpec((NG, H), lambda i: (0, 0))],
        out_shape=(jax.ShapeDtypeStruct((N, H), jnp.float32),
                   jax.ShapeDtypeStruct((NG, H), jnp.float32)),
    )(h, bt, s, cnt, na)

    def body_c(o_ref, bt_ref, v_ref, c_ref, nw_ref, nb_ref, fw_ref, fb_ref,
               x_ref):
        Bm = _onehot(bt_ref[...], _TN)
        cnt = jnp.maximum(c_ref[...], 1.0)
        isd = lax.rsqrt(v_ref[...] / cnt + 1e-5)
        outf = o_ref[...] * _dot1(Bm, isd)
        hn = nw_ref[...][None, :] * outf + nb_ref[...][None, :]
        x_ref[...] = _dot(hn, fw_ref[...]) + fb_ref[...][None, :]

    return pl.pallas_call(
        body_c,
        grid=(N // _TN,),
        in_specs=[pl.BlockSpec((_TN, H), lambda i: (i, 0)),
                  pl.BlockSpec((_TN, 1), lambda i: (i, 0)),
                  pl.BlockSpec((NG, H), lambda i: (0, 0)),
                  pl.BlockSpec((NG, 1), lambda i: (0, 0)),
                  pl.BlockSpec((H,), lambda i: (0,)),
                  pl.BlockSpec((H,), lambda i: (0,)),
                  pl.BlockSpec((H, H), lambda i: (0, 0)),
                  pl.BlockSpec((H,), lambda i: (0,))],
        out_specs=pl.BlockSpec((_TN, H), lambda i: (i, 0)),
        out_shape=jax.ShapeDtypeStruct((N, H), jnp.float32),
    )(out, bt, v, cnt, nw, nb, fw, fb)


def _stage_d(x, batch, w):
    def body(x_ref, bt_ref, w0, b0, w1, b1, w2, b2, ow, ob, o_ref):
        i = pl.program_id(0)
        x = x_ref[...]
        for wref, bref in ((w0, b0), (w1, b1), (w2, b2)):
            x = _swish(_dot(x, wref[...])
                       + bref[...][None, :])
        xo = _dot(x, ow[...]) + ob[...][None, :]
        Bm = _onehot(bt_ref[...], _TN)

        @pl.when(i == 0)
        def _():
            o_ref[...] = jnp.zeros_like(o_ref)

        o_ref[...] += _dotT(Bm, xo)

    wfull = pl.BlockSpec((H, H), lambda i: (0, 0))
    wvec = pl.BlockSpec((H,), lambda i: (0,))
    return pl.pallas_call(
        body,
        grid=(N // _TN,),
        in_specs=[pl.BlockSpec((_TN, H), lambda i: (i, 0)),
                  pl.BlockSpec((_TN, 1), lambda i: (i, 0)),
                  wfull, wvec, wfull, wvec, wfull, wvec,
                  pl.BlockSpec((H, 1), lambda i: (0, 0)),
                  pl.BlockSpec((1,), lambda i: (0,))],
        out_specs=pl.BlockSpec((NG, 1), lambda i: (0, 0)),
        out_shape=jax.ShapeDtypeStruct((NG, 1), jnp.float32),
    )(x, batch, *w)


def _sc_gather(x, src3):
    mesh = plsc.VectorSubcoreMesh(core_axis_name="c", subcore_axis_name="s")

    @functools.partial(
        pl.kernel, mesh=mesh,
        out_type=jax.ShapeDtypeStruct((EP, HH), jnp.int32),
        scratch_types=[
            pltpu.VMEM((CH,), jnp.int32),
            pltpu.VMEM((CH, HH), jnp.int32),
            pltpu.SemaphoreType.DMA,
        ],
    )
    def k(x_hbm, src_hbm, out_hbm, idx_v, rows_v, sem):
        wid = lax.axis_index("s") * NC + lax.axis_index("c")

        def body(i, carry):
            pltpu.sync_copy(src_hbm.at[wid, i], idx_v)
            pltpu.async_copy(x_hbm.at[idx_v], rows_v, sem).wait()
            base = (wid * GCP + i) * CH
            pltpu.sync_copy(rows_v, out_hbm.at[pl.ds(base, CH)])
            return carry

        lax.fori_loop(0, GCP, body, 0)

    return k(x, src3)


def _sc_scatter(m, dst3):
    mesh = plsc.VectorSubcoreMesh(core_axis_name="c", subcore_axis_name="s")

    @functools.partial(
        pl.kernel, mesh=mesh,
        out_type=jax.ShapeDtypeStruct((NC, NP, HH), jnp.float32),
        scratch_types=[
            pltpu.VMEM((CH,), jnp.int32),
            pltpu.VMEM((CH, HH), jnp.float32),
            pltpu.VMEM((ZR, HH), jnp.float32),
            pltpu.VMEM_SHARED((NP, HH), jnp.float32),
        ],
    )
    def k(m_hbm, dst_hbm, out_hbm, idx_v, buf_v, st_v, acc_sh):
        c = lax.axis_index("c")
        s = lax.axis_index("s")

        def zrow(i, carry):
            def zcol(j, carry2):
                st_v[i, pl.ds(j * 16, 16)] = jnp.zeros((16,), jnp.float32)
                return carry2
            return lax.fori_loop(0, HH // 16, zcol, carry)

        lax.fori_loop(0, ZR, zrow, 0)
        for r in range(RPS // ZR):
            pltpu.sync_copy(st_v, acc_sh.at[pl.ds(s * RPS + r * ZR, ZR)])
        plsc.subcore_barrier()

        def body(i, carry):
            pltpu.sync_copy(dst_hbm.at[s, i], idx_v)
            base = (s * SCP + i) * CH
            pltpu.sync_copy(m_hbm.at[c, pl.ds(base, CH)], buf_v)
            pltpu.sync_copy(buf_v, acc_sh.at[idx_v], add=True)
            return carry

        lax.fori_loop(0, SCP, body, 0)
        plsc.subcore_barrier()

        for r in range(RPS // ZR):
            row = s * RPS + r * ZR
            pltpu.sync_copy(acc_sh.at[pl.ds(row, ZR)], st_v)
            pltpu.sync_copy(st_v, out_hbm.at[c, pl.ds(row, ZR)])

    return k(m, dst3)


def kernel(z, feature1, feature2, edge_index, batch, params):
    src = edge_index[0].astype(jnp.int32)
    dst = edge_index[1].astype(jnp.int32)
    pad = EP - E
    src3 = jnp.pad(src, (0, pad)).reshape(NW, GCP, CH)
    dst3 = jnp.pad(dst, (0, pad)).reshape(NS, SCP, CH)
    f1p = jnp.pad(feature1, ((0, pad), (0, 0)))
    f2p = jnp.pad(feature2, ((0, pad), (0, 0)))

    bt2 = batch.astype(jnp.int32).reshape(N, 1)
    x = _prologue(z.astype(jnp.int32).reshape(N, 1), params['emb'])

    for b in params['blocks']:
        xnew, xbf, c1t, c2t = _stage_a(
            x, b['lin_W'].T, b['lin_b'],
            b['f1_W1'].T, b['f1_W2'].T, b['f2_W1'].T, b['f2_W2'].T)
        xb32 = lax.bitcast_convert_type(
            jnp.stack([xbf[:, :HH], xbf[:, HH:]], axis=-1), jnp.int32)
        xg = _sc_gather(xb32, src3)
        m1, m2 = _stage_b(f1p, f2p, xg, c1t, c2t)
        a1 = _sc_scatter(m1, dst3)[:, :N, :]
        a2 = _sc_scatter(m2, dst3)[:, :N, :]
        wlist = [
            b['c1_Wrel'].T, b['c1_brel'], b['c1_Wroot'].T,
            b['lin1_W'].T, b['lin1_b'],
            b['c2_Wrel'].T, b['c2_brel'], b['c2_Wroot'].T,
            b['lin2_W'].T, b['lin2_b'],
            b['cat_W'][:, :H].T, b['cat_W'][:, H:].T, b['cat_b'],
            b['lins'][0][0].T, b['lins'][0][1],
            b['lins'][1][0].T, b['lins'][1][1],
            b['lins'][2][0].T, b['lins'][2][1],
        ]
        h = _stage_c1(xnew, a1, a2, wlist)
        x = _stage_c2(h, bt2,
                      b['norm_w'], b['norm_b'], b['norm_a'],
                      b['final_W'].T, b['final_b'])

    wout = [
        params['out_lins'][0][0].T, params['out_lins'][0][1],
        params['out_lins'][1][0].T, params['out_lins'][1][1],
        params['out_lins'][2][0].T, params['out_lins'][2][1],
        params['out_W'].T, params['out_b'],
    ]
    return _stage_d(x, bt2, wout)
